# baseline (speedup 1.0000x reference)
import jax
import jax.numpy as jnp
from jax import lax
from jax.experimental import pallas as pl
from jax.experimental.pallas import tpu as pltpu
from jax.experimental.pallas import tpu_sc as plsc

N_ROWS = 4096
N_K = 20
N_CLASSES = 1000
N_WORKERS = 32
ROWS_PER_W = N_ROWS // N_WORKERS
G = 2
STEPS = ROWS_PER_W // G
LABELS_PER_STEP = G * N_K
LAB_PER_W = ROWS_PER_W * N_K
_WIN = (0, 16, 24)


def _sc_body(labels_hbm, a_hbm, k_hbm, ones_hbm, zeros16_hbm, zerosblk_hbm,
             out_hbm, lab_v, a_v, k_v, ones_v, zer_v, bufs, sems):
    wid = lax.axis_index("s") * 2 + lax.axis_index("c")
    lab_base = wid * LAB_PER_W
    pltpu.sync_copy(labels_hbm.at[pl.ds(lab_base, LAB_PER_W)], lab_v)
    pltpu.sync_copy(a_hbm.at[pl.ds(lab_base, LAB_PER_W)], a_v)
    pltpu.sync_copy(k_hbm.at[pl.ds(lab_base, LAB_PER_W)], k_v)
    pltpu.sync_copy(ones_hbm, ones_v)
    pltpu.sync_copy(zeros16_hbm, zer_v)
    pltpu.sync_copy(zerosblk_hbm, bufs)

    def _scatter(t, vals):
        slot = t % 2
        for w in _WIN:
            off = t * LABELS_PER_STEP + w
            plsc.store_scatter(
                bufs.at[slot],
                [a_v[pl.ds(off, 16)], k_v[pl.ds(off, 16)],
                 lab_v[pl.ds(off, 16)]],
                vals[...],
            )

    def _dma(t):
        return pltpu.make_async_copy(
            bufs.at[t % 2],
            out_hbm.at[pl.ds(wid * ROWS_PER_W + t * G, G)],
            sems.at[t % 2],
        )

    for t in range(STEPS):
        if t >= 2:
            _dma(t - 2).wait()
            _scatter(t - 2, zer_v)
        _scatter(t, ones_v)
        _dma(t).start()
    _dma(STEPS - 2).wait()
    _dma(STEPS - 1).wait()


_sc_onehot = pl.kernel(
    _sc_body,
    out_type=jax.ShapeDtypeStruct((N_ROWS, N_K, N_CLASSES), jnp.float32),
    mesh=plsc.VectorSubcoreMesh(core_axis_name="c", subcore_axis_name="s"),
    compiler_params=pltpu.CompilerParams(needs_layout_passes=False),
    scratch_types=[
        pltpu.VMEM((LAB_PER_W,), jnp.int32),
        pltpu.VMEM((LAB_PER_W,), jnp.int32),
        pltpu.VMEM((LAB_PER_W,), jnp.int32),
        pltpu.VMEM((16,), jnp.float32),
        pltpu.VMEM((16,), jnp.float32),
        pltpu.VMEM((2, G, N_K, N_CLASSES), jnp.float32),
        pltpu.SemaphoreType.DMA((2,)),
    ],
)


def kernel(labels):
    labels_flat = labels.reshape(N_ROWS * N_K)
    m = jnp.arange(N_ROWS * N_K, dtype=jnp.int32)
    a_idx = (m // N_K) % G
    k_idx = m % N_K
    ones16 = jnp.ones((16,), jnp.float32)
    zeros16 = jnp.zeros((16,), jnp.float32)
    zeros_blk = jnp.zeros((2, G, N_K, N_CLASSES), jnp.float32)
    return _sc_onehot(labels_flat, a_idx, k_idx, ones16, zeros16, zeros_blk)


# speedup vs baseline: 1.3370x; 1.3370x over previous
import jax
import jax.numpy as jnp
from jax.experimental import pallas as pl

ROW_BLOCK = 256


def _onehot_block(labels_ref, out_ref):
    labels = labels_ref[...]
    iota = jax.lax.broadcasted_iota(jnp.int32, (1, 1, 1024), 2)
    out_ref[...] = (labels[:, :, None] == iota).astype(jnp.float32)


def kernel(labels):
    n, k = labels.shape
    labels_pad = jnp.pad(labels, ((0, 0), (0, 24 - k)), constant_values=-1)
    grid = (n // ROW_BLOCK,)
    big = pl.pallas_call(
        _onehot_block,
        grid=grid,
        in_specs=[pl.BlockSpec((ROW_BLOCK, 24), lambda i: (i, 0))],
        out_specs=pl.BlockSpec((ROW_BLOCK, 24, 1024), lambda i: (i, 0, 0)),
        out_shape=jax.ShapeDtypeStruct((n, 24, 1024), jnp.float32),
    )(labels_pad)
    return big[:, :k, :1000]


# speedup vs baseline: 1.3520x; 1.0112x over previous
import jax
import jax.numpy as jnp
from jax.experimental import pallas as pl

ROW_BLOCK = 128


def _onehot_block(labels_ref, out_ref):
    labels = labels_ref[...]
    iota = jax.lax.broadcasted_iota(jnp.int32, (1, 1, 1024), 2)
    out_ref[...] = (labels[:, :, None] == iota).astype(jnp.float32)


def kernel(labels):
    n, k = labels.shape
    labels_pad = jnp.pad(labels, ((0, 0), (0, 24 - k)), constant_values=-1)
    grid = (n // ROW_BLOCK,)
    big = pl.pallas_call(
        _onehot_block,
        grid=grid,
        in_specs=[pl.BlockSpec((ROW_BLOCK, 24), lambda i: (i, 0))],
        out_specs=pl.BlockSpec((ROW_BLOCK, 24, 1024), lambda i: (i, 0, 0)),
        out_shape=jax.ShapeDtypeStruct((n, 24, 1024), jnp.float32),
    )(labels_pad)
    return big[:, :k, :1000]
